# Initial kernel scaffold; baseline (speedup 1.0000x reference)
import jax
import jax.numpy as jnp
from jax.experimental import pallas as pl


def kernel(features, table, W1, b1, W2, b2):
    raise NotImplementedError("write your pallas kernel here")


# speedup vs baseline: 8.3358x; 8.3358x over previous
import functools

import jax
import jax.numpy as jnp
from jax import lax
from jax.experimental import pallas as pl
from jax.experimental.pallas import tpu as pltpu
from jax.experimental.pallas import tpu_sc as plsc

VOCAB = 1000000
D = 128
B = 16384
L = 20
BL = B * L

NC = 2
NS = 16
NW = NC * NS
ROWS_PER_W = BL // NW
CHUNK = 128
NCHUNK = ROWS_PER_W // CHUNK


def _gather_body(feat_hbm, table_hbm, emb_hbm, idx_v, rows_v, gsem):
    wid = lax.axis_index("s") * NC + lax.axis_index("c")
    base = wid * ROWS_PER_W
    pltpu.sync_copy(feat_hbm.at[wid], idx_v)

    def step(j, _):
        pltpu.async_copy(table_hbm.at[idx_v.at[j]], rows_v, gsem).wait()
        pltpu.sync_copy(rows_v, emb_hbm.at[pl.ds(base + j * CHUNK, CHUNK)])
        return _

    lax.fori_loop(0, NCHUNK, step, None)


def _sc_gather(table, feat):
    mesh = plsc.VectorSubcoreMesh(core_axis_name="c", subcore_axis_name="s")
    k = pl.kernel(
        _gather_body,
        mesh=mesh,
        out_type=jax.ShapeDtypeStruct((BL, D), jnp.float32),
        scratch_types=[
            pltpu.VMEM((NCHUNK, CHUNK), jnp.int32),
            pltpu.VMEM((CHUNK, D), jnp.float32),
            pltpu.SemaphoreType.DMA,
        ],
    )
    return k(feat, table)


ROWS_BLK = 2048
GRID = BL // ROWS_BLK


def _mlp_body(emb_ref, w1_ref, b1_ref, w2_ref, b2_ref, out_ref):
    h = jnp.dot(emb_ref[...], w1_ref[...], preferred_element_type=jnp.float32)
    h = jnp.maximum(h + b1_ref[...], 0.0)
    o = jnp.dot(h, w2_ref[...], preferred_element_type=jnp.float32)
    out_ref[...] = jnp.maximum(o + b2_ref[...], 0.0)


def _tc_mlp(emb, W1, b1, W2, b2):
    return pl.pallas_call(
        _mlp_body,
        grid=(GRID,),
        in_specs=[
            pl.BlockSpec((ROWS_BLK, D), lambda i: (i, 0)),
            pl.BlockSpec((D, 256), lambda i: (0, 0)),
            pl.BlockSpec((1, 256), lambda i: (0, 0)),
            pl.BlockSpec((256, D), lambda i: (0, 0)),
            pl.BlockSpec((1, D), lambda i: (0, 0)),
        ],
        out_specs=pl.BlockSpec((ROWS_BLK, D), lambda i: (i, 0)),
        out_shape=jax.ShapeDtypeStruct((BL, D), jnp.float32),
    )(emb, W1, b1, W2, b2)


def kernel(features, table, W1, b1, W2, b2):
    feat = features.reshape(-1).astype(jnp.int32).reshape(NW, NCHUNK, CHUNK)
    emb = _sc_gather(table, feat)
    out = _tc_mlp(emb, W1, b1.reshape(1, 256), W2, b2.reshape(1, 128))
    return out.reshape(B, L, D)


# speedup vs baseline: 11.5677x; 1.3877x over previous
import functools

import jax
import jax.numpy as jnp
from jax import lax
from jax.experimental import pallas as pl
from jax.experimental.pallas import tpu as pltpu
from jax.experimental.pallas import tpu_sc as plsc

VOCAB = 1000000
D = 128
B = 16384
L = 20
BL = B * L

NC = 2
NS = 16
NW = NC * NS
ROWS_PER_W = BL // NW
CHUNK = 128
NCHUNK = ROWS_PER_W // CHUNK


def _gather_body(feat_hbm, table_hbm, emb_hbm, idx_v, rows_v, gsem):
    wid = lax.axis_index("s") * NC + lax.axis_index("c")
    base = wid * ROWS_PER_W
    pltpu.sync_copy(feat_hbm.at[wid], idx_v)

    def step(j, _):
        pltpu.async_copy(table_hbm.at[idx_v.at[j]], rows_v, gsem).wait()
        pltpu.sync_copy(rows_v, emb_hbm.at[pl.ds(base + j * CHUNK, CHUNK)])
        return _

    lax.fori_loop(0, NCHUNK, step, None)


def _sc_gather(table, feat):
    mesh = plsc.VectorSubcoreMesh(core_axis_name="c", subcore_axis_name="s")
    k = pl.kernel(
        _gather_body,
        mesh=mesh,
        out_type=jax.ShapeDtypeStruct((BL, D), jnp.float32),
        scratch_types=[
            pltpu.VMEM((NCHUNK, CHUNK), jnp.int32),
            pltpu.VMEM((CHUNK, D), jnp.float32),
            pltpu.SemaphoreType.DMA,
        ],
    )
    return k(feat, table)


B_BLK = 128
ROWS_BLK = B_BLK * L
GRID = B // B_BLK


def _mlp_body(emb_ref, w1_ref, b1_ref, w2_ref, b2_ref, out_ref):
    h = jnp.dot(emb_ref[...], w1_ref[...], preferred_element_type=jnp.float32)
    h = jnp.maximum(h + b1_ref[...], 0.0)
    o = jnp.dot(h, w2_ref[...], preferred_element_type=jnp.float32)
    o = jnp.maximum(o + b2_ref[...], 0.0)
    out_ref[...] = o.reshape(B_BLK, L, D)


def _tc_mlp(emb, W1, b1, W2, b2):
    return pl.pallas_call(
        _mlp_body,
        grid=(GRID,),
        in_specs=[
            pl.BlockSpec((ROWS_BLK, D), lambda i: (i, 0)),
            pl.BlockSpec((D, 256), lambda i: (0, 0)),
            pl.BlockSpec((1, 256), lambda i: (0, 0)),
            pl.BlockSpec((256, D), lambda i: (0, 0)),
            pl.BlockSpec((1, D), lambda i: (0, 0)),
        ],
        out_specs=pl.BlockSpec((B_BLK, L, D), lambda i: (i, 0, 0)),
        out_shape=jax.ShapeDtypeStruct((B, L, D), jnp.float32),
    )(emb, W1, b1, W2, b2)


def kernel(features, table, W1, b1, W2, b2):
    feat = features.reshape(-1).astype(jnp.int32).reshape(NW, NCHUNK, CHUNK)
    emb = _sc_gather(table, feat)
    return _tc_mlp(emb, W1, b1.reshape(1, 256), W2, b2.reshape(1, 128))


# speedup vs baseline: 14.7464x; 1.2748x over previous
import functools

import jax
import jax.numpy as jnp
from jax import lax
from jax.experimental import pallas as pl
from jax.experimental.pallas import tpu as pltpu
from jax.experimental.pallas import tpu_sc as plsc

VOCAB = 1000000
D = 128
B = 16384
L = 20
BL = B * L

NC = 2
NS = 16
NW = NC * NS
ROWS_PER_W = BL // NW
CHUNK = 128
NCHUNK = ROWS_PER_W // CHUNK


def _gather_body(feat_hbm, table_hbm, emb_hbm, idx_v, rows_v, gsem):
    wid = lax.axis_index("s") * NC + lax.axis_index("c")
    base = wid * ROWS_PER_W
    pltpu.sync_copy(feat_hbm.at[wid], idx_v)

    def step(j, _):
        pltpu.async_copy(table_hbm.at[idx_v.at[j]], rows_v, gsem).wait()
        pltpu.sync_copy(rows_v, emb_hbm.at[pl.ds(base + j * CHUNK, CHUNK)])
        return _

    lax.fori_loop(0, NCHUNK, step, None)


def _sc_gather(table, feat):
    mesh = plsc.VectorSubcoreMesh(core_axis_name="c", subcore_axis_name="s")
    k = pl.kernel(
        _gather_body,
        mesh=mesh,
        out_type=jax.ShapeDtypeStruct((BL, D), jnp.float32),
        scratch_types=[
            pltpu.VMEM((NCHUNK, CHUNK), jnp.int32),
            pltpu.VMEM((CHUNK, D), jnp.float32),
            pltpu.SemaphoreType.DMA,
        ],
    )
    return k(feat, table)


ROWS_BLK = 2048
BLKS_PER_L = B // ROWS_BLK
GRID = L * BLKS_PER_L


def _mlp_body(emb_ref, w1_ref, b1_ref, w2_ref, b2_ref, out_ref):
    h = jnp.dot(emb_ref[...], w1_ref[...], preferred_element_type=jnp.float32)
    h = jnp.maximum(h + b1_ref[...], 0.0)
    o = jnp.dot(h, w2_ref[...], preferred_element_type=jnp.float32)
    o = jnp.maximum(o + b2_ref[...], 0.0)
    out_ref[...] = o.reshape(1, ROWS_BLK, D)


def _tc_mlp(emb_t, W1, b1, W2, b2):
    return pl.pallas_call(
        _mlp_body,
        grid=(GRID,),
        in_specs=[
            pl.BlockSpec((ROWS_BLK, D), lambda i: (i, 0)),
            pl.BlockSpec((D, 256), lambda i: (0, 0)),
            pl.BlockSpec((1, 256), lambda i: (0, 0)),
            pl.BlockSpec((256, D), lambda i: (0, 0)),
            pl.BlockSpec((1, D), lambda i: (0, 0)),
        ],
        out_specs=pl.BlockSpec(
            (1, ROWS_BLK, D),
            lambda i: (i // BLKS_PER_L, i % BLKS_PER_L, 0),
        ),
        out_shape=jax.ShapeDtypeStruct((L, B, D), jnp.float32),
    )(emb_t, W1, b1, W2, b2)


def kernel(features, table, W1, b1, W2, b2):
    feat = features.T.reshape(-1).astype(jnp.int32).reshape(NW, NCHUNK, CHUNK)
    emb_t = _sc_gather(table, feat)
    out_t = _tc_mlp(emb_t, W1, b1.reshape(1, 256), W2, b2.reshape(1, 128))
    return jnp.transpose(out_t, (1, 0, 2))


# speedup vs baseline: 19.0049x; 1.2888x over previous
import jax
import jax.numpy as jnp
from jax import lax
from jax.experimental import pallas as pl
from jax.experimental.pallas import tpu as pltpu
from jax.experimental.pallas import tpu_sc as plsc

VOCAB = 1000000
D = 128
B = 16384
L = 20
BL = B * L

NC = 2
NS = 16
NW = NC * NS

NCH = 4
L_PER = L // NCH
CH_ROWS = BL // NCH
ROWS_PER_W = CH_ROWS // NW
CHUNK = 128
NCHUNK = ROWS_PER_W // CHUNK


def _gather_body(feat_hbm, table_hbm, emb_hbm, idx_v, rows_v, gsem):
    wid = lax.axis_index("s") * NC + lax.axis_index("c")
    base = wid * ROWS_PER_W
    pltpu.sync_copy(feat_hbm.at[wid], idx_v)

    def step(j, _):
        pltpu.async_copy(table_hbm.at[idx_v.at[j]], rows_v, gsem).wait()
        pltpu.sync_copy(rows_v, emb_hbm.at[pl.ds(base + j * CHUNK, CHUNK)])
        return _

    lax.fori_loop(0, NCHUNK, step, None)


def _sc_gather(table, feat_chunk):
    mesh = plsc.VectorSubcoreMesh(core_axis_name="c", subcore_axis_name="s")
    k = pl.kernel(
        _gather_body,
        mesh=mesh,
        out_type=jax.ShapeDtypeStruct((CH_ROWS, D), jnp.float32),
        scratch_types=[
            pltpu.VMEM((NCHUNK, CHUNK), jnp.int32),
            pltpu.VMEM((CHUNK, D), jnp.float32),
            pltpu.SemaphoreType.DMA,
        ],
    )
    return k(feat_chunk, table)


ROWS_BLK = 2048
BLKS_PER_L = B // ROWS_BLK
CH_GRID = CH_ROWS // ROWS_BLK


def _mlp_body_first(emb_ref, w1_ref, b1_ref, w2_ref, b2_ref, out_ref):
    h = jnp.dot(emb_ref[...], w1_ref[...], preferred_element_type=jnp.float32)
    h = jnp.maximum(h + b1_ref[...], 0.0)
    o = jnp.dot(h, w2_ref[...], preferred_element_type=jnp.float32)
    o = jnp.maximum(o + b2_ref[...], 0.0)
    out_ref[...] = o.reshape(1, ROWS_BLK, D)


def _mlp_body_chained(emb_ref, w1_ref, b1_ref, w2_ref, b2_ref, carry_ref,
                      out_ref):
    del carry_ref
    _mlp_body_first(emb_ref, w1_ref, b1_ref, w2_ref, b2_ref, out_ref)


def _tc_mlp_chunk(c, emb_c, W1, b1, W2, b2, carry):
    weight_specs = [
        pl.BlockSpec((D, 256), lambda i: (0, 0)),
        pl.BlockSpec((1, 256), lambda i: (0, 0)),
        pl.BlockSpec((256, D), lambda i: (0, 0)),
        pl.BlockSpec((1, D), lambda i: (0, 0)),
    ]
    in_specs = [pl.BlockSpec((ROWS_BLK, D), lambda i: (i, 0))] + weight_specs
    args = [emb_c, W1, b1, W2, b2]
    kwargs = {}
    body = _mlp_body_first
    if carry is not None:
        in_specs.append(pl.BlockSpec(memory_space=pl.ANY))
        args.append(carry)
        kwargs["input_output_aliases"] = {5: 0}
        body = _mlp_body_chained
    return pl.pallas_call(
        body,
        grid=(CH_GRID,),
        in_specs=in_specs,
        out_specs=pl.BlockSpec(
            (1, ROWS_BLK, D),
            lambda i, c=c: (c * L_PER + i // BLKS_PER_L, i % BLKS_PER_L, 0),
        ),
        out_shape=jax.ShapeDtypeStruct((L, B, D), jnp.float32),
        **kwargs,
    )(*args)


def kernel(features, table, W1, b1, W2, b2):
    feat = features.T.reshape(-1).astype(jnp.int32)
    feat = feat.reshape(NCH, NW, NCHUNK, CHUNK)
    b1r = b1.reshape(1, 256)
    b2r = b2.reshape(1, 128)
    embs = [_sc_gather(table, feat[c]) for c in range(NCH)]
    out = None
    for c in range(NCH):
        out = _tc_mlp_chunk(c, embs[c], W1, b1r, W2, b2r, out)
    return jnp.transpose(out, (1, 0, 2))


# speedup vs baseline: 20.1536x; 1.0604x over previous
import jax
import jax.numpy as jnp
from jax import lax
from jax.experimental import pallas as pl
from jax.experimental.pallas import tpu as pltpu
from jax.experimental.pallas import tpu_sc as plsc

VOCAB = 1000000
D = 128
B = 16384
L = 20
BL = B * L

NC = 2
NS = 16
NW = NC * NS

NCH = 4
L_PER = L // NCH
CH_ROWS = BL // NCH
ROWS_PER_W = CH_ROWS // NW
CHUNK = 128
NCHUNK = ROWS_PER_W // CHUNK


def _gather_body(feat_hbm, table_hbm, emb_hbm, idx_v, rows_v, gsem):
    wid = lax.axis_index("s") * NC + lax.axis_index("c")
    base = wid * ROWS_PER_W
    pltpu.sync_copy(feat_hbm.at[wid], idx_v)

    def step(j, _):
        pltpu.async_copy(table_hbm.at[idx_v.at[j]], rows_v, gsem).wait()
        pltpu.sync_copy(rows_v, emb_hbm.at[pl.ds(base + j * CHUNK, CHUNK)])
        return _

    lax.fori_loop(0, NCHUNK, step, None)


def _sc_gather(table, feat_chunk):
    mesh = plsc.VectorSubcoreMesh(core_axis_name="c", subcore_axis_name="s")
    k = pl.kernel(
        _gather_body,
        mesh=mesh,
        out_type=jax.ShapeDtypeStruct((CH_ROWS, D), jnp.float32),
        scratch_types=[
            pltpu.VMEM((NCHUNK, CHUNK), jnp.int32),
            pltpu.VMEM((CHUNK, D), jnp.float32),
            pltpu.SemaphoreType.DMA,
        ],
    )
    return k(feat_chunk, table)


ROWS_BLK = 4096
BLKS_PER_L = B // ROWS_BLK
CH_GRID = CH_ROWS // ROWS_BLK


def _mlp_body_first(emb_ref, w1_ref, b1_ref, w2_ref, b2_ref, out_ref):
    h = jnp.dot(emb_ref[...], w1_ref[...], preferred_element_type=jnp.float32)
    h = jnp.maximum(h + b1_ref[...], 0.0)
    o = jnp.dot(h, w2_ref[...], preferred_element_type=jnp.float32)
    o = jnp.maximum(o + b2_ref[...], 0.0)
    out_ref[...] = o.reshape(1, ROWS_BLK, D)


def _mlp_body_chained(emb_ref, w1_ref, b1_ref, w2_ref, b2_ref, carry_ref,
                      out_ref):
    del carry_ref
    _mlp_body_first(emb_ref, w1_ref, b1_ref, w2_ref, b2_ref, out_ref)


def _tc_mlp_chunk(c, emb_c, W1, b1, W2, b2, carry):
    weight_specs = [
        pl.BlockSpec((D, 256), lambda i: (0, 0)),
        pl.BlockSpec((1, 256), lambda i: (0, 0)),
        pl.BlockSpec((256, D), lambda i: (0, 0)),
        pl.BlockSpec((1, D), lambda i: (0, 0)),
    ]
    in_specs = [pl.BlockSpec((ROWS_BLK, D), lambda i: (i, 0))] + weight_specs
    args = [emb_c, W1, b1, W2, b2]
    kwargs = {}
    body = _mlp_body_first
    if carry is not None:
        in_specs.append(pl.BlockSpec(memory_space=pl.ANY))
        args.append(carry)
        kwargs["input_output_aliases"] = {5: 0}
        body = _mlp_body_chained
    return pl.pallas_call(
        body,
        grid=(CH_GRID,),
        in_specs=in_specs,
        out_specs=pl.BlockSpec(
            (1, ROWS_BLK, D),
            lambda i, c=c: (c * L_PER + i // BLKS_PER_L, i % BLKS_PER_L, 0),
        ),
        out_shape=jax.ShapeDtypeStruct((L, B, D), jnp.float32),
        **kwargs,
    )(*args)


def kernel(features, table, W1, b1, W2, b2):
    feat = features.T.reshape(-1).astype(jnp.int32)
    feat = feat.reshape(NCH, NW, NCHUNK, CHUNK)
    b1r = b1.reshape(1, 256)
    b2r = b2.reshape(1, 128)
    embs = [_sc_gather(table, feat[c]) for c in range(NCH)]
    out = None
    for c in range(NCH):
        out = _tc_mlp_chunk(c, embs[c], W1, b1r, W2, b2r, out)
    return jnp.transpose(out, (1, 0, 2))
